# baseline (speedup 1.0000x reference)
import functools

import jax
import jax.numpy as jnp
from jax import lax
from jax.experimental import pallas as pl
from jax.experimental.pallas import tpu as pltpu
from jax.experimental.pallas import tpu_sc as plsc

_INFO = plsc.get_sparse_core_info()
_NC = _INFO.num_cores
_NS = _INFO.num_subcores
_NW = _NC * _NS

_ROWS_PER_CHUNK = 2
_RING = 8
_WDEPTH = 2


def _build_gather(n_total: int, d: int, n_chunks: int, rows_per_chunk: int,
                  ring: int, wdepth: int):
    n_per_w = n_total // _NW
    assert n_per_w == n_chunks * rows_per_chunk
    assert n_chunks % ring == 0
    assert 1 <= wdepth < ring
    n_groups = n_chunks // ring
    mesh = plsc.VectorSubcoreMesh(core_axis_name="c", subcore_axis_name="s")

    @functools.partial(
        pl.kernel,
        mesh=mesh,
        out_type=jax.ShapeDtypeStruct((n_total, d), jnp.float32),
        scratch_types=(
            [pltpu.VMEM((n_chunks, rows_per_chunk), jnp.int32)]
            + [pltpu.VMEM((rows_per_chunk, d), jnp.float32) for _ in range(ring)]
            + [pltpu.VMEM_SHARED((_NS, ring // 2, rows_per_chunk, d), jnp.float32)]
            + [pltpu.SemaphoreType.DMA for _ in range(3 * ring)]
        ),
    )
    def gather_kernel(idx_hbm, table_hbm, out_hbm, idx_v, *bufs_and_sems):
        bufs = list(bufs_and_sems[:ring])
        spm = bufs_and_sems[ring]
        gsem = list(bufs_and_sems[ring + 1:2 * ring + 1])
        wsem = list(bufs_and_sems[2 * ring + 1:3 * ring + 1])
        hsem = list(bufs_and_sems[3 * ring + 1:])
        sid = lax.axis_index("s")
        wid = sid * _NC + lax.axis_index("c")
        base = wid * n_per_w

        pltpu.sync_copy(idx_hbm.at[wid], idx_v)

        def issue_gather(chunk, slot):
            pltpu.async_copy(table_hbm.at[idx_v.at[chunk]], bufs[slot],
                             gsem[slot])

        def wait_gather(slot):
            pltpu.make_async_copy(table_hbm.at[idx_v.at[0]], bufs[slot],
                                  gsem[slot]).wait()

        def out_slice(chunk):
            return out_hbm.at[pl.ds(base + chunk * rows_per_chunk,
                                    rows_per_chunk)]

        def issue_write(chunk, slot):
            if slot % 2 == 0:
                pltpu.async_copy(bufs[slot], out_slice(chunk), wsem[slot])
            else:
                @pl.when(chunk >= ring)
                def _():
                    pltpu.make_async_copy(spm.at[sid, slot // 2], out_slice(0),
                                          hsem[slot]).wait()
                pltpu.async_copy(bufs[slot], spm.at[sid, slot // 2], wsem[slot])

        def wait_write(slot, chunk=None):
            pltpu.make_async_copy(bufs[slot],
                                  out_hbm.at[pl.ds(base, rows_per_chunk)],
                                  wsem[slot]).wait()
            if slot % 2 == 1 and chunk is not None:
                pltpu.async_copy(spm.at[sid, slot // 2], out_slice(chunk),
                                 hsem[slot])

        lead = ring - wdepth

        for b in range(lead):
            issue_gather(b, b)

        def group(gr, _):
            for b in range(ring):
                chunk = gr * ring + b
                wait_gather(b)
                issue_write(chunk, b)
                slot = (b - wdepth) % ring
                if b < wdepth:
                    @pl.when(gr > 0)
                    def _():
                        wait_write(slot, chunk - wdepth)
                else:
                    wait_write(slot, chunk - wdepth)
                refill = chunk + lead
                if b < wdepth:
                    issue_gather(refill, slot)
                else:
                    @pl.when(refill < n_chunks)
                    def _():
                        issue_gather(refill, slot)
            return ()

        lax.fori_loop(0, n_groups, group, (), unroll=False)

        for b in range(ring - wdepth, ring):
            wait_write(b, n_chunks - wdepth + (b - (ring - wdepth)))
        for b in range(1, ring, 2):
            pltpu.make_async_copy(spm.at[sid, b // 2],
                                  out_hbm.at[pl.ds(base, rows_per_chunk)],
                                  hsem[b]).wait()

    return gather_kernel


def kernel(indices, table):
    b, s = indices.shape
    v, d = table.shape
    n_total = b * s
    n_chunks = n_total // _NW // _ROWS_PER_CHUNK
    idx = jnp.asarray(indices, jnp.int32).reshape(_NW, n_chunks,
                                                  _ROWS_PER_CHUNK)
    gather = _build_gather(n_total, d, n_chunks, _ROWS_PER_CHUNK, _RING,
                           _WDEPTH)
    out = gather(idx, table)
    return out.reshape(b, s, d)


# speedup vs baseline: 1.0020x; 1.0020x over previous
import functools

import jax
import jax.numpy as jnp
from jax import lax
from jax.experimental import pallas as pl
from jax.experimental.pallas import tpu as pltpu
from jax.experimental.pallas import tpu_sc as plsc

_INFO = plsc.get_sparse_core_info()
_NC = _INFO.num_cores
_NS = _INFO.num_subcores
_NW = _NC * _NS

_ROWS_PER_CHUNK = 4
_RING = 4
_WDEPTH = 2


def _build_gather(n_total: int, d: int, n_chunks: int, rows_per_chunk: int,
                  ring: int, wdepth: int):
    n_per_w = n_total // _NW
    assert n_per_w == n_chunks * rows_per_chunk
    assert n_chunks % ring == 0
    assert 1 <= wdepth < ring
    n_groups = n_chunks // ring
    mesh = plsc.VectorSubcoreMesh(core_axis_name="c", subcore_axis_name="s")

    @functools.partial(
        pl.kernel,
        mesh=mesh,
        out_type=jax.ShapeDtypeStruct((n_total, d), jnp.float32),
        scratch_types=(
            [pltpu.VMEM((n_chunks, rows_per_chunk), jnp.int32)]
            + [pltpu.VMEM((rows_per_chunk, d), jnp.float32) for _ in range(ring)]
            + [pltpu.VMEM_SHARED((_NS, ring // 2, rows_per_chunk, d), jnp.float32)]
            + [pltpu.SemaphoreType.DMA for _ in range(3 * ring)]
        ),
    )
    def gather_kernel(idx_hbm, table_hbm, out_hbm, idx_v, *bufs_and_sems):
        bufs = list(bufs_and_sems[:ring])
        spm = bufs_and_sems[ring]
        gsem = list(bufs_and_sems[ring + 1:2 * ring + 1])
        wsem = list(bufs_and_sems[2 * ring + 1:3 * ring + 1])
        hsem = list(bufs_and_sems[3 * ring + 1:])
        sid = lax.axis_index("s")
        wid = sid * _NC + lax.axis_index("c")
        base = wid * n_per_w

        pltpu.sync_copy(idx_hbm.at[wid], idx_v)

        def issue_gather(chunk, slot):
            pltpu.async_copy(table_hbm.at[idx_v.at[chunk]], bufs[slot],
                             gsem[slot])

        def wait_gather(slot):
            pltpu.make_async_copy(table_hbm.at[idx_v.at[0]], bufs[slot],
                                  gsem[slot]).wait()

        def out_slice(chunk):
            return out_hbm.at[pl.ds(base + chunk * rows_per_chunk,
                                    rows_per_chunk)]

        def issue_write(chunk, slot):
            if slot % 2 == 0:
                pltpu.async_copy(bufs[slot], out_slice(chunk), wsem[slot])
            else:
                @pl.when(chunk >= ring)
                def _():
                    pltpu.make_async_copy(spm.at[sid, slot // 2], out_slice(0),
                                          hsem[slot]).wait()
                pltpu.async_copy(bufs[slot], spm.at[sid, slot // 2], wsem[slot])

        def wait_write(slot, chunk=None):
            pltpu.make_async_copy(bufs[slot],
                                  out_hbm.at[pl.ds(base, rows_per_chunk)],
                                  wsem[slot]).wait()
            if slot % 2 == 1 and chunk is not None:
                pltpu.async_copy(spm.at[sid, slot // 2], out_slice(chunk),
                                 hsem[slot])

        lead = ring - wdepth

        for b in range(lead):
            issue_gather(b, b)

        def group(gr, _):
            for b in range(ring):
                chunk = gr * ring + b
                wait_gather(b)
                issue_write(chunk, b)
                slot = (b - wdepth) % ring
                if b < wdepth:
                    @pl.when(gr > 0)
                    def _():
                        wait_write(slot, chunk - wdepth)
                else:
                    wait_write(slot, chunk - wdepth)
                refill = chunk + lead
                if b < wdepth:
                    issue_gather(refill, slot)
                else:
                    @pl.when(refill < n_chunks)
                    def _():
                        issue_gather(refill, slot)
            return ()

        lax.fori_loop(0, n_groups, group, (), unroll=False)

        for b in range(ring - wdepth, ring):
            wait_write(b, n_chunks - wdepth + (b - (ring - wdepth)))
        for b in range(1, ring, 2):
            pltpu.make_async_copy(spm.at[sid, b // 2],
                                  out_hbm.at[pl.ds(base, rows_per_chunk)],
                                  hsem[b]).wait()

    return gather_kernel


def kernel(indices, table):
    b, s = indices.shape
    v, d = table.shape
    n_total = b * s
    n_chunks = n_total // _NW // _ROWS_PER_CHUNK
    idx = jnp.asarray(indices, jnp.int32).reshape(_NW, n_chunks,
                                                  _ROWS_PER_CHUNK)
    gather = _build_gather(n_total, d, n_chunks, _ROWS_PER_CHUNK, _RING,
                           _WDEPTH)
    out = gather(idx, table)
    return out.reshape(b, s, d)


# speedup vs baseline: 1.0142x; 1.0122x over previous
import functools

import jax
import jax.numpy as jnp
from jax import lax
from jax.experimental import pallas as pl
from jax.experimental.pallas import tpu as pltpu
from jax.experimental.pallas import tpu_sc as plsc

_INFO = plsc.get_sparse_core_info()
_NC = _INFO.num_cores
_NS = _INFO.num_subcores
_NW = _NC * _NS

_ROWS_PER_CHUNK = 4
_RING = 4
_WDEPTH = 1


def _build_gather(n_total: int, d: int, n_chunks: int, rows_per_chunk: int,
                  ring: int, wdepth: int):
    n_per_w = n_total // _NW
    assert n_per_w == n_chunks * rows_per_chunk
    assert n_chunks % ring == 0
    assert 1 <= wdepth < ring
    n_groups = n_chunks // ring
    mesh = plsc.VectorSubcoreMesh(core_axis_name="c", subcore_axis_name="s")

    @functools.partial(
        pl.kernel,
        mesh=mesh,
        out_type=jax.ShapeDtypeStruct((n_total, d), jnp.float32),
        scratch_types=(
            [pltpu.VMEM((n_chunks, rows_per_chunk), jnp.int32)]
            + [pltpu.VMEM((rows_per_chunk, d), jnp.float32) for _ in range(ring)]
            + [pltpu.VMEM_SHARED((_NS, ring // 2, rows_per_chunk, d), jnp.float32)]
            + [pltpu.SemaphoreType.DMA for _ in range(3 * ring)]
        ),
    )
    def gather_kernel(idx_hbm, table_hbm, out_hbm, idx_v, *bufs_and_sems):
        bufs = list(bufs_and_sems[:ring])
        spm = bufs_and_sems[ring]
        gsem = list(bufs_and_sems[ring + 1:2 * ring + 1])
        wsem = list(bufs_and_sems[2 * ring + 1:3 * ring + 1])
        hsem = list(bufs_and_sems[3 * ring + 1:])
        sid = lax.axis_index("s")
        wid = sid * _NC + lax.axis_index("c")
        base = wid * n_per_w

        pltpu.sync_copy(idx_hbm.at[wid], idx_v)

        def issue_gather(chunk, slot):
            pltpu.async_copy(table_hbm.at[idx_v.at[chunk]], bufs[slot],
                             gsem[slot])

        def wait_gather(slot):
            pltpu.make_async_copy(table_hbm.at[idx_v.at[0]], bufs[slot],
                                  gsem[slot]).wait()

        def out_slice(chunk):
            return out_hbm.at[pl.ds(base + chunk * rows_per_chunk,
                                    rows_per_chunk)]

        def issue_write(chunk, slot):
            if slot % 2 == 0:
                pltpu.async_copy(bufs[slot], out_slice(chunk), wsem[slot])
            else:
                @pl.when(chunk >= ring)
                def _():
                    pltpu.make_async_copy(spm.at[sid, slot // 2], out_slice(0),
                                          hsem[slot]).wait()
                pltpu.async_copy(bufs[slot], spm.at[sid, slot // 2], wsem[slot])

        def wait_write(slot, chunk=None):
            pltpu.make_async_copy(bufs[slot],
                                  out_hbm.at[pl.ds(base, rows_per_chunk)],
                                  wsem[slot]).wait()
            if slot % 2 == 1 and chunk is not None:
                pltpu.async_copy(spm.at[sid, slot // 2], out_slice(chunk),
                                 hsem[slot])

        lead = ring - wdepth

        for b in range(lead):
            issue_gather(b, b)

        def group(gr, _):
            for b in range(ring):
                chunk = gr * ring + b
                wait_gather(b)
                issue_write(chunk, b)
                slot = (b - wdepth) % ring
                if b < wdepth:
                    @pl.when(gr > 0)
                    def _():
                        wait_write(slot, chunk - wdepth)
                else:
                    wait_write(slot, chunk - wdepth)
                refill = chunk + lead
                if b < wdepth:
                    issue_gather(refill, slot)
                else:
                    @pl.when(refill < n_chunks)
                    def _():
                        issue_gather(refill, slot)
            return ()

        lax.fori_loop(0, n_groups, group, (), unroll=False)

        for b in range(ring - wdepth, ring):
            wait_write(b, n_chunks - wdepth + (b - (ring - wdepth)))
        for b in range(1, ring, 2):
            pltpu.make_async_copy(spm.at[sid, b // 2],
                                  out_hbm.at[pl.ds(base, rows_per_chunk)],
                                  hsem[b]).wait()

    return gather_kernel


def kernel(indices, table):
    b, s = indices.shape
    v, d = table.shape
    n_total = b * s
    n_chunks = n_total // _NW // _ROWS_PER_CHUNK
    idx = jnp.asarray(indices, jnp.int32).reshape(_NW, n_chunks,
                                                  _ROWS_PER_CHUNK)
    gather = _build_gather(n_total, d, n_chunks, _ROWS_PER_CHUNK, _RING,
                           _WDEPTH)
    out = gather(idx, table)
    return out.reshape(b, s, d)
